# Initial kernel scaffold; baseline (speedup 1.0000x reference)
import jax
import jax.numpy as jnp
from jax.experimental import pallas as pl


def kernel(word_ids, char_ids, W_words, W_chars):
    raise NotImplementedError("write your pallas kernel here")


# speedup vs baseline: 1.3908x; 1.3908x over previous
import functools

import jax
import jax.numpy as jnp
from jax import lax
from jax.experimental import pallas as pl
from jax.experimental.pallas import tpu as pltpu
from jax.experimental.pallas import tpu_sc as plsc

VOCAB = 1000000
D = 32
BATCH = 4096
SEQ = 200
N = BATCH * SEQ

NUM_CORES = 2
NUM_SUBCORES = 16
NW = NUM_CORES * NUM_SUBCORES
ROWS_PER_W = N // NW
CHUNK = 256
NCHUNKS = ROWS_PER_W // CHUNK
GSUB = 128
NSUB = CHUNK // GSUB
L = 16

_mesh = plsc.VectorSubcoreMesh(core_axis_name="c", subcore_axis_name="s")


@functools.partial(
    pl.kernel,
    mesh=_mesh,
    compiler_params=pltpu.CompilerParams(use_tc_tiling_on_sc=False),
    out_type=jax.ShapeDtypeStruct((N, 2 * D), jnp.float32),
    scratch_types=[
        pltpu.VMEM((CHUNK,), jnp.int32),
        pltpu.VMEM((CHUNK,), jnp.int32),
        pltpu.VMEM((CHUNK, D), jnp.float32),
        pltpu.VMEM((CHUNK, D), jnp.float32),
        pltpu.VMEM((CHUNK, 2 * D), jnp.float32),
        pltpu.SemaphoreType.DMA,
    ],
)
def _embed_cat(wid_hbm, cid_hbm, ww_hbm, wc_hbm, out_hbm,
               widx_v, cidx_v, wrow_v, crow_v, comb_v, sem):
    w = lax.axis_index("s") * NUM_CORES + lax.axis_index("c")
    base = w * ROWS_PER_W

    def body(i, _):
        off = base + i * CHUNK
        pltpu.sync_copy(wid_hbm.at[pl.ds(off, CHUNK)], widx_v)
        pltpu.sync_copy(cid_hbm.at[pl.ds(off, CHUNK)], cidx_v)
        copies = []
        for j in range(CHUNK // L):
            sl = pl.ds(j * L, L)
            copies.append(pltpu.async_copy(ww_hbm.at[widx_v[sl]], wrow_v.at[sl], sem))
            copies.append(pltpu.async_copy(wc_hbm.at[cidx_v[sl]], crow_v.at[sl], sem))
        for c in copies:
            c.wait()
        def irow(r, _):
            comb_v[r, pl.ds(0, L)] = wrow_v[r, pl.ds(0, L)]
            comb_v[r, pl.ds(L, L)] = wrow_v[r, pl.ds(L, L)]
            comb_v[r, pl.ds(2 * L, L)] = crow_v[r, pl.ds(0, L)]
            comb_v[r, pl.ds(3 * L, L)] = crow_v[r, pl.ds(L, L)]
            return ()
        lax.fori_loop(0, CHUNK, irow, ())
        pltpu.sync_copy(comb_v, out_hbm.at[pl.ds(off, CHUNK)])
        return ()

    lax.fori_loop(0, NCHUNKS, body, ())


def kernel(word_ids, char_ids, W_words, W_chars):
    wid = word_ids.reshape(N).astype(jnp.int32)
    cid = char_ids.reshape(N).astype(jnp.int32)
    out = _embed_cat(wid, cid, W_words, W_chars)
    return out.reshape(BATCH, SEQ, 2 * D)


# speedup vs baseline: 1.6961x; 1.2195x over previous
import functools

import jax
import jax.numpy as jnp
from jax import lax
from jax.experimental import pallas as pl
from jax.experimental.pallas import tpu as pltpu
from jax.experimental.pallas import tpu_sc as plsc

VOCAB = 1000000
D = 32
BATCH = 4096
SEQ = 200
N = BATCH * SEQ

NUM_CORES = 2
NUM_SUBCORES = 16
NW = NUM_CORES * NUM_SUBCORES
ROWS_PER_W = N // NW
CHUNK = 256
NCHUNKS = ROWS_PER_W // CHUNK
GSUB = 128
NSUB = CHUNK // GSUB
L = 16

_mesh = plsc.VectorSubcoreMesh(core_axis_name="c", subcore_axis_name="s")


@functools.partial(
    pl.kernel,
    mesh=_mesh,
    compiler_params=pltpu.CompilerParams(use_tc_tiling_on_sc=False),
    out_type=jax.ShapeDtypeStruct((N, 2 * D), jnp.float32),
    scratch_types=[
        pltpu.VMEM((CHUNK,), jnp.int32),
        pltpu.VMEM((CHUNK,), jnp.int32),
        pltpu.VMEM((CHUNK, D), jnp.float32),
        pltpu.VMEM((CHUNK, D), jnp.float32),
        pltpu.VMEM((CHUNK, 2 * D), jnp.float32),
        pltpu.SemaphoreType.DMA,
    ],
)
def _embed_cat(wid_hbm, cid_hbm, ww_hbm, wc_hbm, out_hbm,
               widx_v, cidx_v, wrow_v, crow_v, comb_v, sem):
    w = lax.axis_index("s") * NUM_CORES + lax.axis_index("c")
    base = w * ROWS_PER_W

    def body(i, _):
        off = base + i * CHUNK
        pltpu.sync_copy(wid_hbm.at[pl.ds(off, CHUNK)], widx_v)
        pltpu.sync_copy(cid_hbm.at[pl.ds(off, CHUNK)], cidx_v)
        copies = []
        for j in range(CHUNK // L):
            sl = pl.ds(j * L, L)
            copies.append(pltpu.async_copy(ww_hbm.at[widx_v[sl]], wrow_v.at[sl], sem))
            copies.append(pltpu.async_copy(wc_hbm.at[cidx_v[sl]], crow_v.at[sl], sem))
        for c in copies:
            c.wait()
        pltpu.sync_copy(wrow_v, out_hbm.at[pl.ds(off, CHUNK), pl.ds(0, D)])
        pltpu.sync_copy(crow_v, out_hbm.at[pl.ds(off, CHUNK), pl.ds(D, D)])
        return ()

    lax.fori_loop(0, NCHUNKS, body, ())


def kernel(word_ids, char_ids, W_words, W_chars):
    wid = word_ids.reshape(N).astype(jnp.int32)
    cid = char_ids.reshape(N).astype(jnp.int32)
    out = _embed_cat(wid, cid, W_words, W_chars)
    return out.reshape(BATCH, SEQ, 2 * D)


# speedup vs baseline: 1.8823x; 1.1098x over previous
import functools

import jax
import jax.numpy as jnp
from jax import lax
from jax.experimental import pallas as pl
from jax.experimental.pallas import tpu as pltpu
from jax.experimental.pallas import tpu_sc as plsc

VOCAB = 1000000
D = 32
BATCH = 4096
SEQ = 200
N = BATCH * SEQ

NUM_CORES = 2
NUM_SUBCORES = 16
NW = NUM_CORES * NUM_SUBCORES
ROWS_PER_W = N // NW
CHUNK = 512
NCHUNKS = ROWS_PER_W // CHUNK
NPAIR = NCHUNKS // 2
L = 16
NG = CHUNK // L

_mesh = plsc.VectorSubcoreMesh(core_axis_name="c", subcore_axis_name="s")


@functools.partial(
    pl.kernel,
    mesh=_mesh,
    compiler_params=pltpu.CompilerParams(use_tc_tiling_on_sc=False),
    out_type=jax.ShapeDtypeStruct((N, 2 * D), jnp.float32),
    scratch_types=[
        pltpu.VMEM((ROWS_PER_W,), jnp.int32),
        pltpu.VMEM((ROWS_PER_W,), jnp.int32),
        pltpu.VMEM((CHUNK, D), jnp.float32),
        pltpu.VMEM((CHUNK, D), jnp.float32),
        pltpu.VMEM((CHUNK, D), jnp.float32),
        pltpu.VMEM((CHUNK, D), jnp.float32),
        pltpu.SemaphoreType.DMA,
        pltpu.SemaphoreType.DMA,
        pltpu.SemaphoreType.DMA,
        pltpu.SemaphoreType.DMA,
    ],
)
def _embed_cat(wid_hbm, cid_hbm, ww_hbm, wc_hbm, out_hbm,
               widx_v, cidx_v, wrow0, wrow1, crow0, crow1,
               gsem0, gsem1, wsem0, wsem1):
    w = lax.axis_index("s") * NUM_CORES + lax.axis_index("c")
    base = w * ROWS_PER_W
    wrow = (wrow0, wrow1)
    crow = (crow0, crow1)
    gsem = (gsem0, gsem1)
    wsem = (wsem0, wsem1)

    pltpu.sync_copy(wid_hbm.at[pl.ds(base, ROWS_PER_W)], widx_v)
    pltpu.sync_copy(cid_hbm.at[pl.ds(base, ROWS_PER_W)], cidx_v)

    def enqueue_gathers(i, b):
        for j in range(NG):
            sl = pl.ds(i * CHUNK + j * L, L)
            dst = pl.ds(j * L, L)
            pltpu.async_copy(ww_hbm.at[widx_v[sl]], wrow[b].at[dst], gsem[b])
            pltpu.async_copy(wc_hbm.at[cidx_v[sl]], crow[b].at[dst], gsem[b])

    def drain_gathers(b):
        pltpu.make_async_copy(ww_hbm.at[pl.ds(0, CHUNK)], wrow[b], gsem[b]).wait()
        pltpu.make_async_copy(wc_hbm.at[pl.ds(0, CHUNK)], crow[b], gsem[b]).wait()

    def issue_writes(i, b):
        off = base + i * CHUNK
        pltpu.async_copy(wrow[b], out_hbm.at[pl.ds(off, CHUNK), pl.ds(0, D)], wsem[b])
        pltpu.async_copy(crow[b], out_hbm.at[pl.ds(off, CHUNK), pl.ds(D, D)], wsem[b])

    def wait_writes(b):
        pltpu.make_async_copy(
            wrow[b], out_hbm.at[pl.ds(base, CHUNK), pl.ds(0, D)], wsem[b]).wait()
        pltpu.make_async_copy(
            crow[b], out_hbm.at[pl.ds(base, CHUNK), pl.ds(D, D)], wsem[b]).wait()

    enqueue_gathers(0, 0)
    enqueue_gathers(1, 1)
    drain_gathers(0)
    issue_writes(0, 0)
    drain_gathers(1)
    issue_writes(1, 1)

    def pair(k, _):
        i0 = 2 * k + 2
        wait_writes(0)
        enqueue_gathers(i0, 0)
        wait_writes(1)
        enqueue_gathers(i0 + 1, 1)
        drain_gathers(0)
        issue_writes(i0, 0)
        drain_gathers(1)
        issue_writes(i0 + 1, 1)
        return ()

    lax.fori_loop(0, NPAIR - 1, pair, ())
    wait_writes(0)
    wait_writes(1)


def kernel(word_ids, char_ids, W_words, W_chars):
    wid = word_ids.reshape(N).astype(jnp.int32)
    cid = char_ids.reshape(N).astype(jnp.int32)
    out = _embed_cat(wid, cid, W_words, W_chars)
    return out.reshape(BATCH, SEQ, 2 * D)
